# Initial kernel scaffold; baseline (speedup 1.0000x reference)
import jax
import jax.numpy as jnp
from jax.experimental import pallas as pl


def kernel(pool, ind):
    raise NotImplementedError("write your pallas kernel here")


# speedup vs baseline: 5.3730x; 5.3730x over previous
import jax
import jax.numpy as jnp
from jax import lax
from jax.experimental import pallas as pl
from jax.experimental.pallas import tpu as pltpu
from jax.experimental.pallas import tpu_sc as plsc

B, H, W, C = 4, 112, 112, 96
OUT_FLAT = (H * 2) * (W * 2) * C
N = B * H * W * C
TOTAL = B * OUT_FLAT
NC, NS = 2, 16
PASSES = 6
CHUNK = TOTAL // (NC * PASSES)
TILE_IN = N // NS
KBLK = 2048
NBLK = TILE_IN // KBLK
TILE_OUT = CHUNK // NS
ZBUF = 12544


def _sc_body(pool_hbm, ind_hbm, out_hbm, idx_v, val_v, zero_v, acc_sh):
    c = lax.axis_index("c")
    s = lax.axis_index("s")

    def zfill(i, _):
        zero_v[pl.ds(i * 16, 16)] = jnp.zeros((16,), jnp.float32)
        return 0
    lax.fori_loop(0, ZBUF // 16, zfill, 0)

    in_base = s * TILE_IN
    b_off = (s // 4) * OUT_FLAT
    dump = lax.iota(jnp.int32, 16) * 8 + CHUNK

    for p in range(PASSES):
        chunk_id = p * NC + c
        base = chunk_id * CHUNK
        for z in range(TILE_OUT // ZBUF):
            pltpu.sync_copy(zero_v,
                            acc_sh.at[pl.ds(s * TILE_OUT + z * ZBUF, ZBUF)])
        plsc.subcore_barrier()

        def blk(i, _):
            off = in_base + i * KBLK
            pltpu.sync_copy(ind_hbm.at[pl.ds(off, KBLK)], idx_v)
            pltpu.sync_copy(pool_hbm.at[pl.ds(off, KBLK)], val_v)

            def vx(j, _):
                g = idx_v[pl.ds(j * 16, 16)] + b_off
                m = (g >= base) & (g < base + CHUNK)
                idx_v[pl.ds(j * 16, 16)] = jnp.where(m, g - base, dump)
                return 0
            lax.fori_loop(0, KBLK // 16, vx, 0)
            pltpu.sync_copy(val_v, acc_sh.at[idx_v], add=True)
            return 0
        lax.fori_loop(0, NBLK, blk, 0)
        plsc.subcore_barrier()
        pltpu.sync_copy(acc_sh.at[pl.ds(s * TILE_OUT, TILE_OUT)],
                        out_hbm.at[pl.ds(base + s * TILE_OUT, TILE_OUT)])


def kernel(pool, ind):
    pool_flat = pool.reshape(-1)
    ind_flat = ind.reshape(-1).astype(jnp.int32)
    mesh = plsc.VectorSubcoreMesh(core_axis_name="c", subcore_axis_name="s")
    out = pl.kernel(
        _sc_body,
        out_type=jax.ShapeDtypeStruct((TOTAL,), jnp.float32),
        mesh=mesh,
        scratch_types=[
            pltpu.VMEM((KBLK,), jnp.int32),
            pltpu.VMEM((KBLK,), jnp.float32),
            pltpu.VMEM((ZBUF,), jnp.float32),
            pltpu.VMEM_SHARED((CHUNK + 128,), jnp.float32),
        ],
    )(pool_flat, ind_flat)
    return out.reshape(B, H * 2, W * 2, C)


# speedup vs baseline: 6.3998x; 1.1911x over previous
import jax
import jax.numpy as jnp
from jax import lax
from jax.experimental import pallas as pl
from jax.experimental.pallas import tpu as pltpu
from jax.experimental.pallas import tpu_sc as plsc

B, H, W, C = 4, 112, 112, 96
OUT_FLAT = (H * 2) * (W * 2) * C
N = B * H * W * C
TOTAL = B * OUT_FLAT
NC, NS = 2, 16
PASSES = 6
CHUNK = TOTAL // (NC * PASSES)
TILE_IN = N // NS
KBLK = 1536
NBLK = TILE_IN // KBLK
NBUF = 4
TILE_OUT = CHUNK // NS
ZBUF = 12544


def _sc_body(pool_hbm, ind_hbm, out_hbm,
             idx_v0, idx_v1, idx_v2, idx_v3,
             val_v0, val_v1, val_v2, val_v3,
             zero_v, acc_sh, isem, vsem, ssem):
    idx_v = (idx_v0, idx_v1, idx_v2, idx_v3)
    val_v = (val_v0, val_v1, val_v2, val_v3)
    c = lax.axis_index("c")
    s = lax.axis_index("s")

    def zfill(i, _):
        zero_v[pl.ds(i * 16, 16)] = jnp.zeros((16,), jnp.float32)
        return 0
    lax.fori_loop(0, ZBUF // 16, zfill, 0)

    in_base = s * TILE_IN
    b_off = (s // 4) * OUT_FLAT
    dump = lax.iota(jnp.int32, 16) * 8 + CHUNK

    def start_in(i, b):
        off = in_base + i * KBLK
        pltpu.async_copy(ind_hbm.at[pl.ds(off, KBLK)], idx_v[b], isem.at[b])
        pltpu.async_copy(pool_hbm.at[pl.ds(off, KBLK)], val_v[b], vsem.at[b])

    def wait_in(b):
        pltpu.make_async_copy(ind_hbm.at[pl.ds(0, KBLK)], idx_v[b],
                              isem.at[b]).wait()
        pltpu.make_async_copy(pool_hbm.at[pl.ds(0, KBLK)], val_v[b],
                              vsem.at[b]).wait()

    def fire_sc(b):
        pltpu.async_copy(val_v[b], acc_sh.at[idx_v[b]], ssem.at[b],
                         add=True)

    def wait_sc(b):
        pltpu.make_async_copy(val_v[b], acc_sh.at[idx_v[b]],
                              ssem.at[b]).wait()

    for p in range(PASSES):
        chunk_id = p * NC + c
        base = chunk_id * CHUNK
        shift = b_off - base
        for z in range(TILE_OUT // ZBUF):
            pltpu.sync_copy(zero_v,
                            acc_sh.at[pl.ds(s * TILE_OUT + z * ZBUF, ZBUF)])
        plsc.subcore_barrier()

        start_in(0, 0)
        start_in(1, 1)

        def quad(g, _):
            for b in range(NBUF):
                i = g * NBUF + b
                wait_in(b)

                def vx(j, _):
                    loc = idx_v[b][pl.ds(j * 16, 16)] + shift
                    m = (loc >= 0) & (loc < CHUNK)
                    idx_v[b][pl.ds(j * 16, 16)] = jnp.where(m, loc, dump)
                    return 0
                lax.fori_loop(0, KBLK // 16, vx, 0)

                nb = (b + 2) % NBUF
                @pl.when(i >= 2)
                def _():
                    wait_sc(nb)
                @pl.when(i + 2 <= NBLK - 1)
                def _():
                    start_in(i + 2, nb)
                fire_sc(b)
            return 0
        lax.fori_loop(0, NBLK // NBUF, quad, 0)
        wait_sc((NBLK - 2) % NBUF)
        wait_sc((NBLK - 1) % NBUF)
        plsc.subcore_barrier()
        pltpu.sync_copy(acc_sh.at[pl.ds(s * TILE_OUT, TILE_OUT)],
                        out_hbm.at[pl.ds(base + s * TILE_OUT, TILE_OUT)])


def kernel(pool, ind):
    pool_flat = pool.reshape(-1)
    ind_flat = ind.reshape(-1).astype(jnp.int32)
    mesh = plsc.VectorSubcoreMesh(core_axis_name="c", subcore_axis_name="s")
    out = pl.kernel(
        _sc_body,
        out_type=jax.ShapeDtypeStruct((TOTAL,), jnp.float32),
        mesh=mesh,
        scratch_types=[
            pltpu.VMEM((KBLK,), jnp.int32),
            pltpu.VMEM((KBLK,), jnp.int32),
            pltpu.VMEM((KBLK,), jnp.int32),
            pltpu.VMEM((KBLK,), jnp.int32),
            pltpu.VMEM((KBLK,), jnp.float32),
            pltpu.VMEM((KBLK,), jnp.float32),
            pltpu.VMEM((KBLK,), jnp.float32),
            pltpu.VMEM((KBLK,), jnp.float32),
            pltpu.VMEM((ZBUF,), jnp.float32),
            pltpu.VMEM_SHARED((CHUNK + 128,), jnp.float32),
            pltpu.SemaphoreType.DMA((NBUF,)),
            pltpu.SemaphoreType.DMA((NBUF,)),
            pltpu.SemaphoreType.DMA((NBUF,)),
        ],
    )(pool_flat, ind_flat)
    return out.reshape(B, H * 2, W * 2, C)


# speedup vs baseline: 14.8900x; 2.3266x over previous
import jax
import jax.numpy as jnp
from jax import lax
from jax.experimental import pallas as pl
from jax.experimental.pallas import tpu as pltpu
from jax.experimental.pallas import tpu_sc as plsc

B, H, W, C = 4, 112, 112, 96
OUT_FLAT = (H * 2) * (W * 2) * C
N = B * H * W * C
TOTAL = B * OUT_FLAT
NC, NS = 2, 16
PASSES = 6
CHUNK = TOTAL // (NC * PASSES)
TILE_IN = N // NS
KBLK = 1536
NBLK = TILE_IN // KBLK
NBUF = 4
SEND = 2048
SBUF = SEND + KBLK + 32
TILE_OUT = CHUNK // NS
ZBUF = 2048
GRP = 4


def _sc_body(pool_hbm, ind_hbm, out_hbm,
             idx_v0, idx_v1, idx_v2, idx_v3,
             val_v0, val_v1, val_v2, val_v3,
             sidx, sval, cstash, zero_v, acc_sh, isem, vsem):
    idx_v = (idx_v0, idx_v1, idx_v2, idx_v3)
    val_v = (val_v0, val_v1, val_v2, val_v3)
    c = lax.axis_index("c")
    s = lax.axis_index("s")

    def zfill(i, _):
        zero_v[pl.ds(i * 16, 16)] = jnp.zeros((16,), jnp.float32)
        return 0
    lax.fori_loop(0, ZBUF // 16, zfill, 0)

    in_base = s * TILE_IN
    b_off = (s // 4) * OUT_FLAT
    dump = lax.iota(jnp.int32, 16) * 8 + CHUNK
    lane = lax.iota(jnp.int32, 16)

    def start_in(i, b):
        off = in_base + i * KBLK
        pltpu.async_copy(ind_hbm.at[pl.ds(off, KBLK)], idx_v[b], isem.at[b])
        pltpu.async_copy(pool_hbm.at[pl.ds(off, KBLK)], val_v[b], vsem.at[b])

    def wait_in(b):
        pltpu.make_async_copy(ind_hbm.at[pl.ds(0, KBLK)], idx_v[b],
                              isem.at[b]).wait()
        pltpu.make_async_copy(pool_hbm.at[pl.ds(0, KBLK)], val_v[b],
                              vsem.at[b]).wait()

    def fire(n):
        pltpu.sync_copy(sval.at[pl.ds(0, n)],
                        acc_sh.at[sidx.at[pl.ds(0, n)]], add=True)

    for p in range(PASSES):
        chunk_id = p * NC + c
        base = chunk_id * CHUNK
        shift = b_off - base
        for z in range(TILE_OUT // ZBUF):
            pltpu.sync_copy(zero_v,
                            acc_sh.at[pl.ds(s * TILE_OUT + z * ZBUF, ZBUF)])
        plsc.subcore_barrier()

        start_in(0, 0)
        start_in(1, 1)

        def quad(g, cnt):
            for b in range(NBUF):
                i = g * NBUF + b
                wait_in(b)
                nb = (b + 2) % NBUF
                @pl.when(i + 2 <= NBLK - 1)
                def _():
                    start_in(i + 2, nb)

                def compute_group(t):
                    locs, msks, valss, pops = [], [], [], []
                    for u in range(GRP):
                        off = t * (GRP * 16) + u * 16
                        loc = idx_v[b][pl.ds(off, 16)] + shift
                        m = (loc >= 0) & (loc < CHUNK)
                        locs.append(loc)
                        msks.append(m)
                        valss.append(val_v[b][pl.ds(off, 16)])
                        pops.append(plsc.all_reduce_population_count(m)[0])
                    return tuple(locs), tuple(msks), tuple(valss), tuple(pops)

                def store_group(cnt, locs, msks, valss, pops):
                    for u in range(GRP):
                        plsc.store_compressed(sidx.at[pl.ds(cnt, 16)],
                                              locs[u], mask=msks[u])
                        plsc.store_compressed(sval.at[pl.ds(cnt, 16)],
                                              valss[u], mask=msks[u])
                        cnt = cnt + pops[u]
                    return cnt

                g0 = compute_group(0)

                def vx(t, carry):
                    cnt, locs, msks, valss, pops = carry
                    nxt = compute_group(t)
                    cnt = store_group(cnt, locs, msks, valss, pops)
                    return (cnt,) + nxt
                carry = lax.fori_loop(1, KBLK // (GRP * 16), vx, (cnt,) + g0)
                cnt = store_group(*carry)

                do_fire = cnt >= SEND
                @pl.when(do_fire)
                def _():
                    fire(SEND)
                    nsh = cnt - SEND
                    def sh(k, _):
                        sidx[pl.ds(k * 16, 16)] = sidx[pl.ds(SEND + k * 16, 16)]
                        sval[pl.ds(k * 16, 16)] = sval[pl.ds(SEND + k * 16, 16)]
                        return 0
                    lax.fori_loop(0, (nsh + 15) // 16, sh, 0)
                cnt = jnp.where(do_fire, cnt - SEND, cnt)
            return cnt
        cnt = lax.fori_loop(0, NBLK // NBUF, quad, jnp.int32(0))

        def pad(k, _):
            mpad = (lane + k * 16) >= cnt
            v = sidx[pl.ds(k * 16, 16)]
            sidx[pl.ds(k * 16, 16)] = jnp.where(mpad, dump, v)
            w = sval[pl.ds(k * 16, 16)]
            sval[pl.ds(k * 16, 16)] = jnp.where(mpad, 0.0, w)
            return 0
        lax.fori_loop(0, SEND // 16, pad, 0)
        fire(SEND)

        plsc.subcore_barrier()
        pltpu.sync_copy(acc_sh.at[pl.ds(s * TILE_OUT, TILE_OUT)],
                        out_hbm.at[pl.ds(base + s * TILE_OUT, TILE_OUT)])


def kernel(pool, ind):
    pool_flat = pool.reshape(-1)
    ind_flat = ind.reshape(-1).astype(jnp.int32)
    mesh = plsc.VectorSubcoreMesh(core_axis_name="c", subcore_axis_name="s")
    out = pl.kernel(
        _sc_body,
        out_type=jax.ShapeDtypeStruct((TOTAL,), jnp.float32),
        mesh=mesh,
        compiler_params=pltpu.CompilerParams(needs_layout_passes=False),
        scratch_types=[
            pltpu.VMEM((KBLK,), jnp.int32),
            pltpu.VMEM((KBLK,), jnp.int32),
            pltpu.VMEM((KBLK,), jnp.int32),
            pltpu.VMEM((KBLK,), jnp.int32),
            pltpu.VMEM((KBLK,), jnp.float32),
            pltpu.VMEM((KBLK,), jnp.float32),
            pltpu.VMEM((KBLK,), jnp.float32),
            pltpu.VMEM((KBLK,), jnp.float32),
            pltpu.VMEM((SBUF,), jnp.int32),
            pltpu.VMEM((SBUF,), jnp.float32),
            pltpu.VMEM((16,), jnp.int32),
            pltpu.VMEM((ZBUF,), jnp.float32),
            pltpu.VMEM_SHARED((CHUNK + 128,), jnp.float32),
            pltpu.SemaphoreType.DMA((NBUF,)),
            pltpu.SemaphoreType.DMA((NBUF,)),
        ],
    )(pool_flat, ind_flat)
    return out.reshape(B, H * 2, W * 2, C)


# speedup vs baseline: 15.4555x; 1.0380x over previous
import jax
import jax.numpy as jnp
from jax import lax
from jax.experimental import pallas as pl
from jax.experimental.pallas import tpu as pltpu
from jax.experimental.pallas import tpu_sc as plsc

B, H, W, C = 4, 112, 112, 96
OUT_FLAT = (H * 2) * (W * 2) * C
N = B * H * W * C
TOTAL = B * OUT_FLAT
NC, NS = 2, 16
PASSES = 6
CHUNK = TOTAL // (NC * PASSES)
TILE_IN = N // NS
KBLK = 1792
NBLK = TILE_IN // KBLK
NBUF = 4
SEND = 2048
SBUF = SEND + KBLK + 32
TILE_OUT = CHUNK // NS
ZBUF = 2048
GRP = 8


def _sc_body(pool_hbm, ind_hbm, out_hbm,
             idx_v0, idx_v1, idx_v2, idx_v3,
             val_v0, val_v1, val_v2, val_v3,
             sidx, sval, cstash, zero_v, acc_sh, isem, vsem):
    idx_v = (idx_v0, idx_v1, idx_v2, idx_v3)
    val_v = (val_v0, val_v1, val_v2, val_v3)
    c = lax.axis_index("c")
    s = lax.axis_index("s")

    def zfill(i, _):
        zero_v[pl.ds(i * 16, 16)] = jnp.zeros((16,), jnp.float32)
        return 0
    lax.fori_loop(0, ZBUF // 16, zfill, 0)

    in_base = s * TILE_IN
    b_off = (s // 4) * OUT_FLAT
    dump = lax.iota(jnp.int32, 16) * 8 + CHUNK
    lane = lax.iota(jnp.int32, 16)

    def start_in(i, b):
        off = in_base + i * KBLK
        pltpu.async_copy(ind_hbm.at[pl.ds(off, KBLK)], idx_v[b], isem.at[b])
        pltpu.async_copy(pool_hbm.at[pl.ds(off, KBLK)], val_v[b], vsem.at[b])

    def wait_in(b):
        pltpu.make_async_copy(ind_hbm.at[pl.ds(0, KBLK)], idx_v[b],
                              isem.at[b]).wait()
        pltpu.make_async_copy(pool_hbm.at[pl.ds(0, KBLK)], val_v[b],
                              vsem.at[b]).wait()

    def fire(n):
        pltpu.sync_copy(sval.at[pl.ds(0, n)],
                        acc_sh.at[sidx.at[pl.ds(0, n)]], add=True)

    for p in range(PASSES):
        chunk_id = p * NC + c
        base = chunk_id * CHUNK
        shift = b_off - base
        for z in range(TILE_OUT // ZBUF):
            pltpu.sync_copy(zero_v,
                            acc_sh.at[pl.ds(s * TILE_OUT + z * ZBUF, ZBUF)])
        plsc.subcore_barrier()

        start_in(0, 0)
        start_in(1, 1)

        def quad(g, cnt):
            for b in range(NBUF):
                i = g * NBUF + b
                wait_in(b)
                nb = (b + 2) % NBUF
                @pl.when(i + 2 <= NBLK - 1)
                def _():
                    start_in(i + 2, nb)

                def compute_group(t):
                    locs, msks, valss, pops = [], [], [], []
                    for u in range(GRP):
                        off = t * (GRP * 16) + u * 16
                        loc = idx_v[b][pl.ds(off, 16)] + shift
                        m = (loc >= 0) & (loc < CHUNK)
                        locs.append(loc)
                        msks.append(m)
                        valss.append(val_v[b][pl.ds(off, 16)])
                        pops.append(plsc.all_reduce_population_count(m)[0])
                    return tuple(locs), tuple(msks), tuple(valss), tuple(pops)

                def store_group(cnt, locs, msks, valss, pops):
                    for u in range(GRP):
                        plsc.store_compressed(sidx.at[pl.ds(cnt, 16)],
                                              locs[u], mask=msks[u])
                        plsc.store_compressed(sval.at[pl.ds(cnt, 16)],
                                              valss[u], mask=msks[u])
                        cnt = cnt + pops[u]
                    return cnt

                g0 = compute_group(0)

                def vx(t, carry):
                    cnt, locs, msks, valss, pops = carry
                    nxt = compute_group(t)
                    cnt = store_group(cnt, locs, msks, valss, pops)
                    return (cnt,) + nxt
                carry = lax.fori_loop(1, KBLK // (GRP * 16), vx, (cnt,) + g0)
                cnt = store_group(*carry)

                do_fire = cnt >= SEND
                @pl.when(do_fire)
                def _():
                    fire(SEND)
                    nsh = cnt - SEND
                    def sh(k, _):
                        sidx[pl.ds(k * 16, 16)] = sidx[pl.ds(SEND + k * 16, 16)]
                        sval[pl.ds(k * 16, 16)] = sval[pl.ds(SEND + k * 16, 16)]
                        return 0
                    lax.fori_loop(0, (nsh + 15) // 16, sh, 0)
                cnt = jnp.where(do_fire, cnt - SEND, cnt)
            return cnt
        cnt = lax.fori_loop(0, NBLK // NBUF, quad, jnp.int32(0))

        def pad(k, _):
            mpad = (lane + k * 16) >= cnt
            v = sidx[pl.ds(k * 16, 16)]
            sidx[pl.ds(k * 16, 16)] = jnp.where(mpad, dump, v)
            w = sval[pl.ds(k * 16, 16)]
            sval[pl.ds(k * 16, 16)] = jnp.where(mpad, 0.0, w)
            return 0
        lax.fori_loop(0, SEND // 16, pad, 0)
        fire(SEND)

        plsc.subcore_barrier()
        pltpu.sync_copy(acc_sh.at[pl.ds(s * TILE_OUT, TILE_OUT)],
                        out_hbm.at[pl.ds(base + s * TILE_OUT, TILE_OUT)])


def kernel(pool, ind):
    pool_flat = pool.reshape(-1)
    ind_flat = ind.reshape(-1).astype(jnp.int32)
    mesh = plsc.VectorSubcoreMesh(core_axis_name="c", subcore_axis_name="s")
    out = pl.kernel(
        _sc_body,
        out_type=jax.ShapeDtypeStruct((TOTAL,), jnp.float32),
        mesh=mesh,
        compiler_params=pltpu.CompilerParams(needs_layout_passes=False),
        scratch_types=[
            pltpu.VMEM((KBLK,), jnp.int32),
            pltpu.VMEM((KBLK,), jnp.int32),
            pltpu.VMEM((KBLK,), jnp.int32),
            pltpu.VMEM((KBLK,), jnp.int32),
            pltpu.VMEM((KBLK,), jnp.float32),
            pltpu.VMEM((KBLK,), jnp.float32),
            pltpu.VMEM((KBLK,), jnp.float32),
            pltpu.VMEM((KBLK,), jnp.float32),
            pltpu.VMEM((SBUF,), jnp.int32),
            pltpu.VMEM((SBUF,), jnp.float32),
            pltpu.VMEM((16,), jnp.int32),
            pltpu.VMEM((ZBUF,), jnp.float32),
            pltpu.VMEM_SHARED((CHUNK + 128,), jnp.float32),
            pltpu.SemaphoreType.DMA((NBUF,)),
            pltpu.SemaphoreType.DMA((NBUF,)),
        ],
    )(pool_flat, ind_flat)
    return out.reshape(B, H * 2, W * 2, C)


# speedup vs baseline: 16.8729x; 1.0917x over previous
import jax
import jax.numpy as jnp
from jax import lax
from jax.experimental import pallas as pl
from jax.experimental.pallas import tpu as pltpu
from jax.experimental.pallas import tpu_sc as plsc

B, H, W, C = 4, 112, 112, 96
OUT_FLAT = (H * 2) * (W * 2) * C
N = B * H * W * C
TOTAL = B * OUT_FLAT
NC, NS = 2, 16
PASSES = 6
CHUNK = TOTAL // (NC * PASSES)
TILE_IN = N // NS
KBLK = 1536
NBLK = TILE_IN // KBLK
NBUF = 4
GRP = 8
SEND = 2048
NREG = 3
SBUF = NREG * SEND + KBLK + 32
TILE_OUT = CHUNK // NS


def _sc_body(pool_hbm, ind_hbm, zeros_hbm, out_hbm,
             idx_v0, idx_v1, idx_v2, idx_v3,
             val_v0, val_v1, val_v2, val_v3,
             sidx, sval, acc_sh, isem, vsem, ssem):
    idx_v = (idx_v0, idx_v1, idx_v2, idx_v3)
    val_v = (val_v0, val_v1, val_v2, val_v3)
    c = lax.axis_index("c")
    s = lax.axis_index("s")

    in_base = s * TILE_IN
    b_off = (s // 4) * OUT_FLAT
    dump = lax.iota(jnp.int32, 16) * 8 + CHUNK
    lane = lax.iota(jnp.int32, 16)

    def start_in(i, b):
        off = in_base + i * KBLK
        pltpu.async_copy(ind_hbm.at[pl.ds(off, KBLK)], idx_v[b], isem.at[b])
        pltpu.async_copy(pool_hbm.at[pl.ds(off, KBLK)], val_v[b], vsem.at[b])

    def wait_in(b):
        pltpu.make_async_copy(ind_hbm.at[pl.ds(0, KBLK)], idx_v[b],
                              isem.at[b]).wait()
        pltpu.make_async_copy(pool_hbm.at[pl.ds(0, KBLK)], val_v[b],
                              vsem.at[b]).wait()

    def fire_async(off):
        pltpu.async_copy(sval.at[pl.ds(off, SEND)],
                         acc_sh.at[sidx.at[pl.ds(off, SEND)]], ssem,
                         add=True)

    def wait_fire():
        pltpu.make_async_copy(sval.at[pl.ds(0, SEND)],
                              acc_sh.at[sidx.at[pl.ds(0, SEND)]], ssem).wait()

    for p in range(PASSES):
        chunk_id = p * NC + c
        base = chunk_id * CHUNK
        shift = b_off - base
        pltpu.sync_copy(zeros_hbm,
                        acc_sh.at[pl.ds(s * TILE_OUT, TILE_OUT)])
        plsc.subcore_barrier()

        start_in(0, 0)
        start_in(1, 1)

        def compute_group(b, t):
            locs, msks, valss, pops = [], [], [], []
            for u in range(GRP):
                off = t * (GRP * 16) + u * 16
                loc = idx_v[b][pl.ds(off, 16)] + shift
                m = (loc >= 0) & (loc < CHUNK)
                locs.append(loc)
                msks.append(m)
                valss.append(val_v[b][pl.ds(off, 16)])
                pops.append(plsc.all_reduce_population_count(m)[0])
            return tuple(locs), tuple(msks), tuple(valss), tuple(pops)

        def store_group(pos, locs, msks, valss, pops):
            for u in range(GRP):
                plsc.store_compressed(sidx.at[pl.ds(pos, 16)],
                                      locs[u], mask=msks[u])
                plsc.store_compressed(sval.at[pl.ds(pos, 16)],
                                      valss[u], mask=msks[u])
                pos = pos + pops[u]
            return pos

        def quad(g, carry):
            for b in range(NBUF):
                i = g * NBUF + b
                wait_in(b)
                nb = (b + 2) % NBUF
                @pl.when(i + 2 <= NBLK - 1)
                def _():
                    start_in(i + 2, nb)

                pos, k = carry
                g0 = compute_group(b, 0)

                def vx(t, vc):
                    pos, locs, msks, valss, pops = vc
                    nxt = compute_group(b, t)
                    pos = store_group(pos, locs, msks, valss, pops)
                    return (pos,) + nxt
                vc = lax.fori_loop(1, KBLK // (GRP * 16), vx, (pos,) + g0)
                pos = store_group(*vc)

                newk = pos // SEND
                @pl.when(newk > k)
                def _():
                    fire_async(k * SEND)
                k = jnp.where(newk > k, newk, k)

                do_wrap = pos >= NREG * SEND
                @pl.when(do_wrap)
                def _():
                    for _i in range(NREG):
                        wait_fire()
                    tail = pos - NREG * SEND
                    def mv(q, _):
                        sidx[pl.ds(q * 16, 16)] = \
                            sidx[pl.ds(NREG * SEND + q * 16, 16)]
                        sval[pl.ds(q * 16, 16)] = \
                            sval[pl.ds(NREG * SEND + q * 16, 16)]
                        return 0
                    lax.fori_loop(0, (tail + 15) // 16, mv, 0)
                pos = jnp.where(do_wrap, pos - NREG * SEND, pos)
                k = jnp.where(do_wrap, 0, k)
                carry = (pos, k)
            return carry
        pos, k = lax.fori_loop(0, NBLK // NBUF, quad,
                               (jnp.int32(0), jnp.int32(0)))

        def drain(_q, _):
            wait_fire()
            return 0
        lax.fori_loop(0, k, drain, 0)
        kbase = k * SEND
        def pad(q, _):
            o = kbase + q * 16
            mpad = (lane + o) >= pos
            v = sidx[pl.ds(o, 16)]
            sidx[pl.ds(o, 16)] = jnp.where(mpad, dump, v)
            w = sval[pl.ds(o, 16)]
            sval[pl.ds(o, 16)] = jnp.where(mpad, 0.0, w)
            return 0
        lax.fori_loop(0, SEND // 16, pad, 0)
        fire_async(kbase)
        wait_fire()

        plsc.subcore_barrier()
        pltpu.sync_copy(acc_sh.at[pl.ds(s * TILE_OUT, TILE_OUT)],
                        out_hbm.at[pl.ds(base + s * TILE_OUT, TILE_OUT)])


def kernel(pool, ind):
    pool_flat = pool.reshape(-1)
    ind_flat = ind.reshape(-1).astype(jnp.int32)
    zeros = jnp.zeros((TILE_OUT,), jnp.float32)
    mesh = plsc.VectorSubcoreMesh(core_axis_name="c", subcore_axis_name="s")
    out = pl.kernel(
        _sc_body,
        out_type=jax.ShapeDtypeStruct((TOTAL,), jnp.float32),
        mesh=mesh,
        compiler_params=pltpu.CompilerParams(needs_layout_passes=False),
        scratch_types=[
            pltpu.VMEM((KBLK,), jnp.int32),
            pltpu.VMEM((KBLK,), jnp.int32),
            pltpu.VMEM((KBLK,), jnp.int32),
            pltpu.VMEM((KBLK,), jnp.int32),
            pltpu.VMEM((KBLK,), jnp.float32),
            pltpu.VMEM((KBLK,), jnp.float32),
            pltpu.VMEM((KBLK,), jnp.float32),
            pltpu.VMEM((KBLK,), jnp.float32),
            pltpu.VMEM((SBUF,), jnp.int32),
            pltpu.VMEM((SBUF,), jnp.float32),
            pltpu.VMEM_SHARED((CHUNK + 128,), jnp.float32),
            pltpu.SemaphoreType.DMA((NBUF,)),
            pltpu.SemaphoreType.DMA((NBUF,)),
            pltpu.SemaphoreType.DMA,
        ],
    )(pool_flat, ind_flat, zeros)
    return out.reshape(B, H * 2, W * 2, C)


# speedup vs baseline: 17.7029x; 1.0492x over previous
import jax
import jax.numpy as jnp
from jax import lax
from jax.experimental import pallas as pl
from jax.experimental.pallas import tpu as pltpu
from jax.experimental.pallas import tpu_sc as plsc

B, H, W, C = 4, 112, 112, 96
OUT_FLAT = (H * 2) * (W * 2) * C
N = B * H * W * C
TOTAL = B * OUT_FLAT
NC, NS = 2, 16
PASSES = 6
CHUNK = TOTAL // (NC * PASSES)
TILE_IN = N // NS
KBLK = 3584
NBLK = TILE_IN // KBLK
NBUF = 2
GRP = 8
SEND = 1024
NREG = 4
SBUF = NREG * SEND + KBLK + 32
TILE_OUT = CHUNK // NS


def _sc_body(pool_hbm, ind_hbm, zeros_hbm, out_hbm,
             idx_v0, idx_v1, val_v0, val_v1,
             sidx, sval, acc_sh, isem, vsem, ssem):
    idx_v = (idx_v0, idx_v1)
    val_v = (val_v0, val_v1)
    c = lax.axis_index("c")
    s = lax.axis_index("s")

    in_base = s * TILE_IN
    b_off = (s // 4) * OUT_FLAT
    dump = lax.iota(jnp.int32, 16) * 8 + CHUNK
    lane = lax.iota(jnp.int32, 16)

    def start_in(i, b):
        off = in_base + i * KBLK
        pltpu.async_copy(ind_hbm.at[pl.ds(off, KBLK)], idx_v[b], isem.at[b])
        pltpu.async_copy(pool_hbm.at[pl.ds(off, KBLK)], val_v[b], vsem.at[b])

    def wait_in(b):
        pltpu.make_async_copy(ind_hbm.at[pl.ds(0, KBLK)], idx_v[b],
                              isem.at[b]).wait()
        pltpu.make_async_copy(pool_hbm.at[pl.ds(0, KBLK)], val_v[b],
                              vsem.at[b]).wait()

    def fire_async(off):
        pltpu.async_copy(sval.at[pl.ds(off, SEND)],
                         acc_sh.at[sidx.at[pl.ds(off, SEND)]], ssem,
                         add=True)

    def wait_fire():
        pltpu.make_async_copy(sval.at[pl.ds(0, SEND)],
                              acc_sh.at[sidx.at[pl.ds(0, SEND)]], ssem).wait()

    for p in range(PASSES):
        chunk_id = p * NC + c
        base = chunk_id * CHUNK
        shift = b_off - base
        pltpu.sync_copy(zeros_hbm,
                        acc_sh.at[pl.ds(s * TILE_OUT, TILE_OUT)])
        plsc.subcore_barrier()

        start_in(0, 0)

        def compute_group(b, t):
            locs, msks, valss, pops = [], [], [], []
            for u in range(GRP):
                off = t * (GRP * 16) + u * 16
                loc = idx_v[b][pl.ds(off, 16)] + shift
                m = (loc >= 0) & (loc < CHUNK)
                locs.append(loc)
                msks.append(m)
                valss.append(val_v[b][pl.ds(off, 16)])
                pops.append(plsc.all_reduce_population_count(m)[0])
            return tuple(locs), tuple(msks), tuple(valss), tuple(pops)

        def store_group(pos, locs, msks, valss, pops):
            for u in range(GRP):
                plsc.store_compressed(sidx.at[pl.ds(pos, 16)],
                                      locs[u], mask=msks[u])
                plsc.store_compressed(sval.at[pl.ds(pos, 16)],
                                      valss[u], mask=msks[u])
                pos = pos + pops[u]
            return pos

        def quad(g, carry):
            for b in range(NBUF):
                i = g * NBUF + b
                nb = (b + 1) % NBUF
                @pl.when(i + 1 <= NBLK - 1)
                def _():
                    start_in(i + 1, nb)
                wait_in(b)

                pos, k = carry
                g0 = compute_group(b, 0)

                def vx(t, vc):
                    pos, locs, msks, valss, pops = vc
                    nxt = compute_group(b, t)
                    pos = store_group(pos, locs, msks, valss, pops)
                    return (pos,) + nxt
                vc = lax.fori_loop(1, KBLK // (GRP * 16), vx, (pos,) + g0)
                pos = store_group(*vc)

                newk = pos // SEND
                for f in range(KBLK // SEND + 1):
                    @pl.when((k + f < NREG) & (newk > k + f))
                    def _():
                        fire_async((k + f) * SEND)
                k = jnp.minimum(jnp.maximum(newk, k), NREG)

                do_wrap = pos >= NREG * SEND
                @pl.when(do_wrap)
                def _():
                    for _i in range(NREG):
                        wait_fire()
                    tail = pos - NREG * SEND
                    def mv(q, _):
                        sidx[pl.ds(q * 16, 16)] = \
                            sidx[pl.ds(NREG * SEND + q * 16, 16)]
                        sval[pl.ds(q * 16, 16)] = \
                            sval[pl.ds(NREG * SEND + q * 16, 16)]
                        return 0
                    lax.fori_loop(0, (tail + 15) // 16, mv, 0)
                pos = jnp.where(do_wrap, pos - NREG * SEND, pos)
                k = jnp.where(do_wrap, 0, k)
                carry = (pos, k)
            return carry
        pos, k = lax.fori_loop(0, NBLK // NBUF, quad,
                               (jnp.int32(0), jnp.int32(0)))

        def drain(_q, _):
            wait_fire()
            return 0
        lax.fori_loop(0, k, drain, 0)
        kbase = k * SEND
        def pad(q, _):
            o = kbase + q * 16
            mpad = (lane + o) >= pos
            v = sidx[pl.ds(o, 16)]
            sidx[pl.ds(o, 16)] = jnp.where(mpad, dump, v)
            w = sval[pl.ds(o, 16)]
            sval[pl.ds(o, 16)] = jnp.where(mpad, 0.0, w)
            return 0
        lax.fori_loop(0, SEND // 16, pad, 0)
        fire_async(kbase)
        wait_fire()

        plsc.subcore_barrier()
        pltpu.sync_copy(acc_sh.at[pl.ds(s * TILE_OUT, TILE_OUT)],
                        out_hbm.at[pl.ds(base + s * TILE_OUT, TILE_OUT)])


def kernel(pool, ind):
    pool_flat = pool.reshape(-1)
    ind_flat = ind.reshape(-1).astype(jnp.int32)
    zeros = jnp.zeros((TILE_OUT,), jnp.float32)
    mesh = plsc.VectorSubcoreMesh(core_axis_name="c", subcore_axis_name="s")
    out = pl.kernel(
        _sc_body,
        out_type=jax.ShapeDtypeStruct((TOTAL,), jnp.float32),
        mesh=mesh,
        compiler_params=pltpu.CompilerParams(needs_layout_passes=False),
        scratch_types=[
            pltpu.VMEM((KBLK,), jnp.int32),
            pltpu.VMEM((KBLK,), jnp.int32),
            pltpu.VMEM((KBLK,), jnp.float32),
            pltpu.VMEM((KBLK,), jnp.float32),
            pltpu.VMEM((SBUF,), jnp.int32),
            pltpu.VMEM((SBUF,), jnp.float32),
            pltpu.VMEM_SHARED((CHUNK + 128,), jnp.float32),
            pltpu.SemaphoreType.DMA((NBUF,)),
            pltpu.SemaphoreType.DMA((NBUF,)),
            pltpu.SemaphoreType.DMA,
        ],
    )(pool_flat, ind_flat, zeros)
    return out.reshape(B, H * 2, W * 2, C)
